# Initial kernel scaffold; baseline (speedup 1.0000x reference)
import jax
import jax.numpy as jnp
from jax.experimental import pallas as pl


def kernel(xyz, new_xyz, features):
    raise NotImplementedError("write your pallas kernel here")


# speedup vs baseline: 9.1560x; 9.1560x over previous
import functools

import jax
import jax.numpy as jnp
from jax import lax
from jax.experimental import pallas as pl
from jax.experimental.pallas import tpu as pltpu
from jax.experimental.pallas import tpu_sc as plsc

RADIUS2 = 0.2 * 0.2
NSAMPLE = 32
L = 16

B, S, N, C = 4, 1024, 8192, 128
NWORK = 32
QBLKS = NWORK // B
QPB = S // QBLKS
NCHUNK = N // L
PADW = NSAMPLE + L
OUTB = QPB * NSAMPLE


def _sc_body(xyz_t, new_xyz, feat, out, pxyz, nq, idxall, chbuf, outbuf, sem):
    wid = lax.axis_index("s") * 2 + lax.axis_index("c")
    b = wid // QBLKS
    qblk = wid % QBLKS

    pltpu.sync_copy(xyz_t.at[pl.ds(b * 3 * N, 3 * N)], pxyz)
    pltpu.sync_copy(
        new_xyz.at[pl.ds((b * S + qblk * QPB) * 3, QPB * 3)], nq)

    iota = lax.broadcasted_iota(jnp.int32, (L,), 0)
    zeros = jnp.zeros((L,), jnp.int32)

    def per_query(sl, _):
        q3 = sl * 3
        qx = plsc.load_gather(nq, [jnp.full((L,), q3, jnp.int32)])
        qy = plsc.load_gather(nq, [jnp.full((L,), q3 + 1, jnp.int32)])
        qz = plsc.load_gather(nq, [jnp.full((L,), q3 + 2, jnp.int32)])
        row = sl * PADW
        idxall[pl.ds(row, L)] = zeros

        def chunk(j, cnt):
            px = pxyz[pl.ds(j * L, L)]
            py = pxyz[pl.ds(N + j * L, L)]
            pz = pxyz[pl.ds(2 * N + j * L, L)]
            dx = px - qx
            dy = py - qy
            dz = pz - qz
            d2 = dx * dx + dy * dy + dz * dz
            m = d2 < RADIUS2
            off = jnp.minimum(cnt, NSAMPLE)
            plsc.store_compressed(
                idxall.at[pl.ds(row + off, L)], j * L + iota, mask=m)
            return cnt + jnp.sum(m.astype(jnp.int32))

        cnt = lax.fori_loop(0, NCHUNK, chunk, jnp.int32(0), unroll=4)
        cntc = jnp.minimum(cnt, NSAMPLE)
        first = plsc.load_gather(idxall, [jnp.full((L,), row, jnp.int32)])
        for half in range(2):
            pos = iota + half * L
            v = idxall[pl.ds(row + half * L, L)]
            idxall[pl.ds(row + half * L, L)] = jnp.where(pos < cntc, v, first)
        return 0

    lax.fori_loop(0, QPB, per_query, 0)

    out_base = (b * (3 + C)) * S * NSAMPLE + qblk * OUTB

    def gather_channel(src_ref, src_off, ch, sub_dim):
        def per_q(sl, _):
            if sub_dim is not None:
                qc = plsc.load_gather(
                    nq, [jnp.full((L,), sl * 3 + sub_dim, jnp.int32)])
            for half in range(2):
                iv = idxall[pl.ds(sl * PADW + half * L, L)]
                vals = plsc.load_gather(src_ref, [iv + src_off])
                if sub_dim is not None:
                    vals = vals - qc
                outbuf[pl.ds(sl * NSAMPLE + half * L, L)] = vals
            return 0

        lax.fori_loop(0, QPB, per_q, 0)
        pltpu.sync_copy(
            outbuf, out.at[pl.ds(out_base + ch * S * NSAMPLE, OUTB)])

    for c3 in range(3):
        gather_channel(pxyz, c3 * N, c3, c3)

    def per_feat(cf, _):
        pltpu.async_copy(
            feat.at[pl.ds((b * C + cf) * N, N)], chbuf, sem).wait()
        gather_channel(chbuf, 0, 3 + cf, None)
        return 0

    lax.fori_loop(0, C, per_feat, 0)


@jax.jit
def _run(xyz_tf, new_xyz_f, feat_f):
    mesh = plsc.VectorSubcoreMesh(core_axis_name="c", subcore_axis_name="s")
    kern = pl.kernel(
        _sc_body,
        out_type=jax.ShapeDtypeStruct((B * (3 + C) * S * NSAMPLE,),
                                      jnp.float32),
        mesh=mesh,
        scratch_types=[
            pltpu.VMEM((3 * N,), jnp.float32),
            pltpu.VMEM((QPB * 3,), jnp.float32),
            pltpu.VMEM((QPB * PADW,), jnp.int32),
            pltpu.VMEM((N,), jnp.float32),
            pltpu.VMEM((OUTB,), jnp.float32),
            pltpu.SemaphoreType.DMA,
        ],
        compiler_params=pltpu.CompilerParams(needs_layout_passes=False),
    )
    return kern(xyz_tf, new_xyz_f, feat_f)


def kernel(xyz, new_xyz, features):
    xyz_tf = jnp.transpose(xyz, (0, 2, 1)).reshape(-1)
    new_xyz_f = new_xyz.reshape(-1)
    feat_f = features.reshape(-1)
    out = _run(xyz_tf, new_xyz_f, feat_f)
    return out.reshape(B, 3 + C, S, NSAMPLE)


# speedup vs baseline: 9.2417x; 1.0094x over previous
import jax
import jax.numpy as jnp
from jax import lax
from jax.experimental import pallas as pl
from jax.experimental.pallas import tpu as pltpu
from jax.experimental.pallas import tpu_sc as plsc

RADIUS2 = 0.2 * 0.2
NSAMPLE = 32
L = 16

B, S, N, C = 4, 1024, 8192, 128
NWORK = 32
QBLKS = NWORK // B
QPB = S // QBLKS
NCHUNK = N // L
PADW = NSAMPLE + L
OUTB = QPB * NSAMPLE
CPP = 4
NPASS = C // CPP


def _sc_body(xyz_t, new_xyz, feat, out,
             pxyz, nq, idxall, chbuf0, chbuf1, outbuf0, outbuf1,
             sem_in0, sem_in1, sem_out0, sem_out1):
    wid = lax.axis_index("s") * 2 + lax.axis_index("c")
    b = wid // QBLKS
    qblk = wid % QBLKS

    pltpu.sync_copy(xyz_t.at[pl.ds(b * 3 * N, 3 * N)], pxyz)
    pltpu.sync_copy(
        new_xyz.at[pl.ds((b * S + qblk * QPB) * 3, QPB * 3)], nq)

    iota = lax.broadcasted_iota(jnp.int32, (L,), 0)
    zeros = jnp.zeros((L,), jnp.int32)
    cap = jnp.full((L,), NSAMPLE, jnp.int32)

    def per_query(sl, _):
        q3 = sl * 3
        qx = plsc.load_gather(nq, [jnp.full((L,), q3, jnp.int32)])
        qy = plsc.load_gather(nq, [jnp.full((L,), q3 + 1, jnp.int32)])
        qz = plsc.load_gather(nq, [jnp.full((L,), q3 + 2, jnp.int32)])
        row = sl * PADW
        row_v = jnp.full((L,), row, jnp.int32)
        idxall[pl.ds(row, L)] = zeros

        def chunk(j, cnt):
            px = pxyz[pl.ds(j * L, L)]
            py = pxyz[pl.ds(N + j * L, L)]
            pz = pxyz[pl.ds(2 * N + j * L, L)]
            dx = px - qx
            dy = py - qy
            dz = pz - qz
            d2 = dx * dx + dy * dy + dz * dz
            m = d2 < RADIUS2
            pos = jnp.minimum(cnt, cap) + plsc.cumsum(m.astype(jnp.int32))
            plsc.store_scatter(
                idxall, [row_v + pos - 1], j * L + iota, mask=m)
            return cnt + plsc.all_reduce_population_count(m)

        cnt = lax.fori_loop(0, NCHUNK, chunk, zeros, unroll=8)
        cntc = jnp.minimum(cnt, cap)
        first = plsc.load_gather(idxall, [row_v])
        for half in range(2):
            pos = iota + half * L
            v = idxall[pl.ds(row + half * L, L)]
            idxall[pl.ds(row + half * L, L)] = jnp.where(pos < cntc, v, first)
        return 0

    lax.fori_loop(0, QPB, per_query, 0)

    out_base = (b * (3 + C)) * S * NSAMPLE + qblk * OUTB

    def per_q_xyz(sl, _):
        for half in range(2):
            iv = idxall[pl.ds(sl * PADW + half * L, L)]
            o = sl * NSAMPLE + half * L
            for c3 in range(3):
                qc = plsc.load_gather(
                    nq, [jnp.full((L,), sl * 3 + c3, jnp.int32)])
                vals = plsc.load_gather(pxyz, [iv + c3 * N]) - qc
                outbuf0[pl.ds(c3 * OUTB + o, L)] = vals
        return 0

    lax.fori_loop(0, QPB, per_q_xyz, 0)
    for c3 in range(3):
        pltpu.sync_copy(
            outbuf0.at[pl.ds(c3 * OUTB, OUTB)],
            out.at[pl.ds(out_base + c3 * S * NSAMPLE, OUTB)])

    def start_in(p, chbuf, sem):
        pltpu.async_copy(
            feat.at[pl.ds((b * C + p * CPP) * N, CPP * N)], chbuf, sem)

    def wait_in(chbuf, sem):
        pltpu.make_async_copy(feat.at[pl.ds(0, CPP * N)], chbuf, sem).wait()

    def out_copy(p, c, outbuf, sem):
        ch = 3 + p * CPP + c
        return pltpu.make_async_copy(
            outbuf.at[pl.ds(c * OUTB, OUTB)],
            out.at[pl.ds(out_base + ch * S * NSAMPLE, OUTB)],
            sem)

    def gather_pass(p, chbuf, outbuf):
        def per_q(sl, _):
            for half in range(2):
                iv = idxall[pl.ds(sl * PADW + half * L, L)]
                o = sl * NSAMPLE + half * L
                for c in range(CPP):
                    vals = plsc.load_gather(chbuf, [iv + c * N])
                    outbuf[pl.ds(c * OUTB + o, L)] = vals
            return 0

        lax.fori_loop(0, QPB, per_q, 0)

    start_in(0, chbuf0, sem_in0)
    start_in(1, chbuf1, sem_in1)

    def pass_pair(pp, _):
        p0 = pp * 2
        wait_in(chbuf0, sem_in0)
        @pl.when(pp > 0)
        def _():
            for c in range(CPP):
                out_copy(p0 - 2, c, outbuf0, sem_out0).wait()
        gather_pass(p0, chbuf0, outbuf0)
        @pl.when(pp < NPASS // 2 - 1)
        def _():
            start_in(p0 + 2, chbuf0, sem_in0)
        for c in range(CPP):
            out_copy(p0, c, outbuf0, sem_out0).start()

        wait_in(chbuf1, sem_in1)
        @pl.when(pp > 0)
        def _():
            for c in range(CPP):
                out_copy(p0 - 1, c, outbuf1, sem_out1).wait()
        gather_pass(p0 + 1, chbuf1, outbuf1)
        @pl.when(pp < NPASS // 2 - 1)
        def _():
            start_in(p0 + 3, chbuf1, sem_in1)
        for c in range(CPP):
            out_copy(p0 + 1, c, outbuf1, sem_out1).start()
        return 0

    lax.fori_loop(0, NPASS // 2, pass_pair, 0)
    for c in range(CPP):
        out_copy(NPASS - 2, c, outbuf0, sem_out0).wait()
        out_copy(NPASS - 1, c, outbuf1, sem_out1).wait()


@jax.jit
def _run(xyz_tf, new_xyz_f, feat_f):
    mesh = plsc.VectorSubcoreMesh(core_axis_name="c", subcore_axis_name="s")
    kern = pl.kernel(
        _sc_body,
        out_type=jax.ShapeDtypeStruct((B * (3 + C) * S * NSAMPLE,),
                                      jnp.float32),
        mesh=mesh,
        scratch_types=[
            pltpu.VMEM((3 * N,), jnp.float32),
            pltpu.VMEM((QPB * 3,), jnp.float32),
            pltpu.VMEM((QPB * PADW,), jnp.int32),
            pltpu.VMEM((CPP * N,), jnp.float32),
            pltpu.VMEM((CPP * N,), jnp.float32),
            pltpu.VMEM((CPP * OUTB,), jnp.float32),
            pltpu.VMEM((CPP * OUTB,), jnp.float32),
            pltpu.SemaphoreType.DMA,
            pltpu.SemaphoreType.DMA,
            pltpu.SemaphoreType.DMA,
            pltpu.SemaphoreType.DMA,
        ],
        compiler_params=pltpu.CompilerParams(needs_layout_passes=False),
    )
    return kern(xyz_tf, new_xyz_f, feat_f)


def kernel(xyz, new_xyz, features):
    xyz_tf = jnp.transpose(xyz, (0, 2, 1)).reshape(-1)
    new_xyz_f = new_xyz.reshape(-1)
    feat_f = features.reshape(-1)
    out = _run(xyz_tf, new_xyz_f, feat_f)
    return out.reshape(B, 3 + C, S, NSAMPLE)


# speedup vs baseline: 22.9546x; 2.4838x over previous
import jax
import jax.numpy as jnp
from jax import lax
from jax.experimental import pallas as pl
from jax.experimental.pallas import tpu as pltpu
from jax.experimental.pallas import tpu_sc as plsc

RADIUS2 = 0.2 * 0.2
NSAMPLE = 32
L = 16

B, S, N, C = 4, 1024, 8192, 128
NWORK = 32
QBLKS = NWORK // B
QPB = S // QBLKS
PADW = NSAMPLE + L
OUTB = QPB * NSAMPLE
CPP = 4
NPASS = C // CPP
W16 = N // 16
GQ = 16
NGRP = QPB // GQ
SBLK = 256


def _tc_body(xyzt_ref, nq_ref, w_ref):
    pi = lax.broadcasted_iota(jnp.int32, (128, 8), 0)
    oi = lax.broadcasted_iota(jnp.int32, (128, 8), 1)
    wp = jnp.where(pi // 16 == oi, (1 << (pi % 16)), 0).astype(jnp.float32)
    q = nq_ref[0]
    parts = []
    for c in range(N // 128):
        p = xyzt_ref[0, :, c * 128:(c + 1) * 128]
        d2 = None
        for d in range(3):
            dd = q[:, d:d + 1] - p[d][None, :]
            dd = dd * dd
            d2 = dd if d2 is None else d2 + dd
        m = (d2 < RADIUS2).astype(jnp.float32)
        parts.append(jnp.dot(m, wp, preferred_element_type=jnp.float32))
    w_ref[0] = jnp.concatenate(parts, axis=1).astype(jnp.int32)


def _tc_pack(xyz_t, new_xyz):
    return pl.pallas_call(
        _tc_body,
        grid=(B, S // SBLK),
        in_specs=[
            pl.BlockSpec((1, 3, N), lambda b, s: (b, 0, 0)),
            pl.BlockSpec((1, SBLK, 3), lambda b, s: (b, s, 0)),
        ],
        out_specs=pl.BlockSpec((1, SBLK, W16), lambda b, s: (b, s, 0)),
        out_shape=jax.ShapeDtypeStruct((B, S, W16), jnp.int32),
    )(xyz_t, new_xyz)


def _sc_body(words, new_xyz, xyz_t, feat, out,
             nq, idxall, nzid, nzval, wbuf0, wbuf1,
             chbuf0, chbuf1, outbuf0, outbuf1,
             sem_w0, sem_w1, sem_in0, sem_in1, sem_out0, sem_out1):
    wid = lax.axis_index("s") * 2 + lax.axis_index("c")
    b = wid // QBLKS
    qblk = wid % QBLKS

    pltpu.sync_copy(
        new_xyz.at[pl.ds((b * S + qblk * QPB) * 3, QPB * 3)], nq)

    pltpu.async_copy(
        xyz_t.at[pl.ds(b * 3 * N, 3 * N)],
        chbuf0.at[pl.ds(0, 3 * N)], sem_in0)
    pltpu.async_copy(feat.at[pl.ds(b * C * N, CPP * N)], chbuf1, sem_in1)

    iota = lax.broadcasted_iota(jnp.int32, (L,), 0)
    zeros = jnp.zeros((L,), jnp.int32)
    ones = jnp.full((L,), 1, jnp.int32)
    cap = jnp.full((L,), NSAMPLE, jnp.int32)

    wgrp_base = (b * S + qblk * QPB) * W16

    def start_w(g, wbuf, sem):
        pltpu.async_copy(
            words.at[pl.ds(wgrp_base + g * GQ * W16, GQ * W16)], wbuf, sem)

    def wait_w(wbuf, sem):
        pltpu.make_async_copy(
            words.at[pl.ds(0, GQ * W16)], wbuf, sem).wait()

    def extract_group(g, wbuf):
        def per_query(sq, _):
            sl = g * GQ + sq
            row = sl * PADW
            row_v = jnp.full((L,), row, jnp.int32)
            idxall[pl.ds(row, L)] = zeros

            def wchunk(wc, carry):
                nwv, nws = carry
                w = wbuf[pl.ds(sq * W16 + wc * L, L)]
                mnz = w != 0
                pos = jnp.minimum(nwv, cap) + plsc.cumsum(
                    mnz.astype(jnp.int32))
                plsc.store_scatter(nzid, [pos - 1], wc * L + iota, mask=mnz)
                plsc.store_scatter(nzval, [pos - 1], w, mask=mnz)
                return (nwv + plsc.all_reduce_population_count(mnz),
                        nws + jnp.sum(mnz.astype(jnp.int32)))

            _, nws = lax.fori_loop(0, W16 // L, wchunk,
                                   (zeros, jnp.int32(0)), unroll=4)

            def word(i, cnt):
                iv = jnp.full((L,), i, jnp.int32)
                t = plsc.load_gather(nzid, [iv])
                w = plsc.load_gather(nzval, [iv])
                m = ((w >> iota) & ones) != 0
                pos = jnp.minimum(cnt, cap) + plsc.cumsum(
                    m.astype(jnp.int32))
                plsc.store_scatter(
                    idxall, [row_v + pos - 1], t * L + iota, mask=m)
                return cnt + plsc.all_reduce_population_count(m)

            cnt = lax.fori_loop(0, jnp.minimum(nws, NSAMPLE), word, zeros)

            cntc = jnp.minimum(cnt, cap)
            first = plsc.load_gather(idxall, [row_v])
            for half in range(2):
                pos = iota + half * L
                v = idxall[pl.ds(row + half * L, L)]
                idxall[pl.ds(row + half * L, L)] = jnp.where(
                    pos < cntc, v, first)
            return 0

        lax.fori_loop(0, GQ, per_query, 0)

    start_w(0, wbuf0, sem_w0)
    start_w(1, wbuf1, sem_w1)

    def grp_pair(gp, _):
        g0 = gp * 2
        wait_w(wbuf0, sem_w0)
        extract_group(g0, wbuf0)
        @pl.when(gp < NGRP // 2 - 1)
        def _():
            start_w(g0 + 2, wbuf0, sem_w0)
        wait_w(wbuf1, sem_w1)
        extract_group(g0 + 1, wbuf1)
        @pl.when(gp < NGRP // 2 - 1)
        def _():
            start_w(g0 + 3, wbuf1, sem_w1)
        return 0

    lax.fori_loop(0, NGRP // 2, grp_pair, 0)

    out_base = (b * (3 + C)) * S * NSAMPLE + qblk * OUTB

    pltpu.make_async_copy(
        xyz_t.at[pl.ds(0, 3 * N)], chbuf0.at[pl.ds(0, 3 * N)],
        sem_in0).wait()

    def per_q_xyz(sl, _):
        for half in range(2):
            iv = idxall[pl.ds(sl * PADW + half * L, L)]
            o = sl * NSAMPLE + half * L
            for c3 in range(3):
                qc = plsc.load_gather(
                    nq, [jnp.full((L,), sl * 3 + c3, jnp.int32)])
                vals = plsc.load_gather(chbuf0, [iv + c3 * N]) - qc
                outbuf0[pl.ds(c3 * OUTB + o, L)] = vals
        return 0

    lax.fori_loop(0, QPB, per_q_xyz, 0)
    for c3 in range(3):
        pltpu.async_copy(
            outbuf0.at[pl.ds(c3 * OUTB, OUTB)],
            out.at[pl.ds(out_base + c3 * S * NSAMPLE, OUTB)],
            sem_out0)

    def start_in(p, chbuf, sem):
        pltpu.async_copy(
            feat.at[pl.ds((b * C + p * CPP) * N, CPP * N)], chbuf, sem)

    def wait_in(chbuf, sem):
        pltpu.make_async_copy(feat.at[pl.ds(0, CPP * N)], chbuf, sem).wait()

    def out_copy(p, c, outbuf, sem):
        ch = 3 + p * CPP + c
        return pltpu.make_async_copy(
            outbuf.at[pl.ds(c * OUTB, OUTB)],
            out.at[pl.ds(out_base + ch * S * NSAMPLE, OUTB)],
            sem)

    def gather_pass(p, chbuf, outbuf):
        def per_q(sl, _):
            for half in range(2):
                iv = idxall[pl.ds(sl * PADW + half * L, L)]
                o = sl * NSAMPLE + half * L
                for c in range(CPP):
                    vals = plsc.load_gather(chbuf, [iv + c * N])
                    outbuf[pl.ds(c * OUTB + o, L)] = vals
            return 0

        lax.fori_loop(0, QPB, per_q, 0)

    def pass_pair(pp, _):
        p0 = pp * 2
        wait_in(chbuf1, sem_in1)
        @pl.when(pp == 0)
        def _():
            for c3 in range(3):
                pltpu.make_async_copy(
                    outbuf0.at[pl.ds(c3 * OUTB, OUTB)],
                    out.at[pl.ds(out_base + c3 * S * NSAMPLE, OUTB)],
                    sem_out0).wait()
        @pl.when(pp > 0)
        def _():
            for c in range(CPP):
                out_copy(p0 - 2, c, outbuf0, sem_out0).wait()
        gather_pass(p0, chbuf1, outbuf0)
        @pl.when(pp < NPASS // 2 - 1)
        def _():
            start_in(p0 + 2, chbuf1, sem_in1)
        for c in range(CPP):
            out_copy(p0, c, outbuf0, sem_out0).start()

        wait_in(chbuf0, sem_in0)
        @pl.when(pp > 0)
        def _():
            for c in range(CPP):
                out_copy(p0 - 1, c, outbuf1, sem_out1).wait()
        gather_pass(p0 + 1, chbuf0, outbuf1)
        @pl.when(pp < NPASS // 2 - 1)
        def _():
            start_in(p0 + 3, chbuf0, sem_in0)
        for c in range(CPP):
            out_copy(p0 + 1, c, outbuf1, sem_out1).start()
        return 0

    pltpu.async_copy(
        feat.at[pl.ds((b * C + CPP) * N, CPP * N)], chbuf0, sem_in0)
    lax.fori_loop(0, NPASS // 2, pass_pair, 0)
    for c in range(CPP):
        out_copy(NPASS - 2, c, outbuf0, sem_out0).wait()
        out_copy(NPASS - 1, c, outbuf1, sem_out1).wait()


@jax.jit
def _run(xyz_tf, new_xyz_f, feat_f, words_f):
    mesh = plsc.VectorSubcoreMesh(core_axis_name="c", subcore_axis_name="s")
    kern = pl.kernel(
        _sc_body,
        out_type=jax.ShapeDtypeStruct((B * (3 + C) * S * NSAMPLE,),
                                      jnp.float32),
        mesh=mesh,
        scratch_types=[
            pltpu.VMEM((QPB * 3,), jnp.float32),
            pltpu.VMEM((QPB * PADW,), jnp.int32),
            pltpu.VMEM((PADW,), jnp.int32),
            pltpu.VMEM((PADW,), jnp.int32),
            pltpu.VMEM((GQ * W16,), jnp.int32),
            pltpu.VMEM((GQ * W16,), jnp.int32),
            pltpu.VMEM((CPP * N,), jnp.float32),
            pltpu.VMEM((CPP * N,), jnp.float32),
            pltpu.VMEM((CPP * OUTB,), jnp.float32),
            pltpu.VMEM((CPP * OUTB,), jnp.float32),
            pltpu.SemaphoreType.DMA,
            pltpu.SemaphoreType.DMA,
            pltpu.SemaphoreType.DMA,
            pltpu.SemaphoreType.DMA,
            pltpu.SemaphoreType.DMA,
            pltpu.SemaphoreType.DMA,
        ],
        compiler_params=pltpu.CompilerParams(needs_layout_passes=False),
    )
    return kern(words_f, new_xyz_f, xyz_tf, feat_f)


def kernel(xyz, new_xyz, features):
    xyz_t = jnp.transpose(xyz, (0, 2, 1)) + jnp.float32(0.0)
    new_xyz_f = new_xyz.reshape(-1) + jnp.float32(0.0)
    words = _tc_pack(xyz_t, new_xyz)
    out = _run(xyz_t.reshape(-1), new_xyz_f,
               features.reshape(-1), words.reshape(-1))
    return out.reshape(B, 3 + C, S, NSAMPLE)
